# baseline (speedup 1.0000x reference)
import jax
import jax.numpy as jnp
from jax.experimental import pallas as pl
from jax.experimental.pallas import tpu as pltpu

_IN = 10
_OUT = 5
_TBT = 2048
_NC = 2


def _linear_tiles_kernel(x_ref, wt_ref, b_ref, o_ref):
    t = o_ref.shape[0]
    x2 = x_ref[...].reshape(t * 8, _IN)
    y = jnp.dot(x2, wt_ref[...], preferred_element_type=jnp.float32)
    o_ref[...] = (y + b_ref[...]).reshape(t, 8, _OUT).astype(o_ref.dtype)


def _linear_tiles_alias_kernel(x_ref, wt_ref, b_ref, prev_ref, o_ref):
    del prev_ref
    _linear_tiles_kernel(x_ref, wt_ref, b_ref, o_ref)


def _chunk_call(xc, wt, b2, prev, T, Tc, off):
    cost = pl.CostEstimate(
        flops=2 * Tc * 8 * _IN * _OUT,
        transcendentals=0,
        bytes_accessed=Tc * 2 * 8 * 128 * 4,
    )
    common = dict(
        out_shape=jax.ShapeDtypeStruct((T, 8, _OUT), xc.dtype),
        grid=(pl.cdiv(Tc, _TBT),),
        out_specs=pl.BlockSpec((_TBT, 8, _OUT), lambda i: (i + off, 0, 0)),
        cost_estimate=cost,
        compiler_params=pltpu.CompilerParams(
            dimension_semantics=("parallel",),
        ),
    )
    in_specs = [
        pl.BlockSpec((_TBT, 8, _IN), lambda i: (i, 0, 0)),
        pl.BlockSpec((_IN, _OUT), lambda i: (0, 0)),
        pl.BlockSpec((1, _OUT), lambda i: (0, 0)),
    ]
    if prev is None:
        return pl.pallas_call(
            _linear_tiles_kernel, in_specs=in_specs, **common,
        )(xc, wt, b2)
    in_specs.append(pl.BlockSpec(memory_space=pltpu.MemorySpace.HBM))
    return pl.pallas_call(
        _linear_tiles_alias_kernel, in_specs=in_specs,
        input_output_aliases={3: 0}, **common,
    )(xc, wt, b2, prev)


@jax.jit
def _forward(x, w, b):
    B = x.shape[0]
    Bp = ((B + 7) // 8) * 8
    if Bp != B:
        x = jnp.pad(x, ((0, Bp - B), (0, 0)))
    T = Bp // 8

    wt = w.T.astype(x.dtype)
    b2 = b.reshape(1, _OUT).astype(x.dtype)

    nc = _NC if T % (_NC * _TBT) == 0 else 1
    if nc == 1:
        xv = x.reshape(T, 8, _IN)
        out = _chunk_call(xv, wt, b2, None, T, T, 0)
    else:
        Tc = T // nc
        Bc = Bp // nc
        out = None
        for c in range(nc):
            xc = x[c * Bc:(c + 1) * Bc].reshape(Tc, 8, _IN)
            out = _chunk_call(xc, wt, b2, out, T, Tc, c * (Tc // _TBT))
    return out.reshape(Bp, _OUT)[:B]


def kernel(x, w, b):
    return _forward(x, w, b)


# speedup vs baseline: 1.4255x; 1.4255x over previous
import jax
import jax.numpy as jnp
from jax.experimental import pallas as pl
from jax.experimental.pallas import tpu as pltpu

_IN = 10
_OUT = 5
_TBT = 3072


def _linear_tiles_kernel(x_ref, wt_ref, b_ref, o_ref):
    t = o_ref.shape[0]
    x2 = x_ref[...].reshape(t * 8, _IN)
    y = jnp.dot(x2, wt_ref[...], preferred_element_type=jnp.float32)
    o_ref[...] = (y + b_ref[...]).reshape(t, 8, _OUT).astype(o_ref.dtype)


@jax.jit
def _forward(x, w, b):
    B = x.shape[0]
    Bp = ((B + 7) // 8) * 8
    if Bp != B:
        x = jnp.pad(x, ((0, Bp - B), (0, 0)))
    T = Bp // 8
    xv = x.reshape(T, 8, _IN)

    wt = w.T.astype(x.dtype)
    b2 = b.reshape(1, _OUT).astype(x.dtype)

    cost = pl.CostEstimate(
        flops=2 * Bp * _IN * _OUT,
        transcendentals=0,
        bytes_accessed=T * 2 * 8 * 128 * 4,
    )

    out = pl.pallas_call(
        _linear_tiles_kernel,
        out_shape=jax.ShapeDtypeStruct((T, 8, _OUT), x.dtype),
        grid=(pl.cdiv(T, _TBT),),
        in_specs=[
            pl.BlockSpec((_TBT, 8, _IN), lambda i: (i, 0, 0)),
            pl.BlockSpec((_IN, _OUT), lambda i: (0, 0)),
            pl.BlockSpec((1, _OUT), lambda i: (0, 0)),
        ],
        out_specs=pl.BlockSpec((_TBT, 8, _OUT), lambda i: (i, 0, 0)),
        cost_estimate=cost,
        compiler_params=pltpu.CompilerParams(
            dimension_semantics=("parallel",),
        ),
    )(xv, wt, b2)

    return out.reshape(Bp, _OUT)[:B]


def kernel(x, w, b):
    return _forward(x, w, b)
